# baseline (speedup 1.0000x reference)
import functools

import jax
import jax.numpy as jnp
from jax import lax
from jax.experimental import pallas as pl
from jax.experimental.pallas import tpu as pltpu
from jax.experimental.pallas import tpu_sc as plsc

MAX_DEPTH = 12
N_CLASSES = 10
NF = 128
N_INTERNAL = 2 ** MAX_DEPTH - 1
N_LEAVES = 2 ** MAX_DEPTH
CHUNK = 160
GROUPS = CHUNK // 16
NC, NS = 2, 16
NW = NC * NS


def _body(x_hbm, thr_hbm, feat_hbm, val_hbm, f0_hbm, t0_hbm, out_hbm,
          feat_v, thr_v, val_v, f0_v, t0_v, sh0, sh1, x0, ob0, ob1,
          ld0, ld1):
    n = x_hbm.shape[0]
    nch = n // CHUNK
    njobs = (nch + NW - 1) // NW
    npairs = (njobs + 1) // 2
    sid = lax.axis_index("s")
    wid = sid * NC + lax.axis_index("c")

    pltpu.sync_copy(feat_hbm.at[pl.ds(0, N_LEAVES)], feat_v)
    pltpu.sync_copy(thr_hbm.at[pl.ds(0, N_LEAVES)], thr_v)
    pltpu.sync_copy(val_hbm, val_v)
    pltpu.sync_copy(f0_hbm, f0_v)
    pltpu.sync_copy(t0_hbm, t0_v)
    f0 = f0_v[...]
    t0 = t0_v[...]

    lane = lax.iota(jnp.int32, 16)
    samps = [lane + 16 * g for g in range(GROUPS)]
    pos10 = lane * N_CLASSES

    def process(c, shb, obuf):
        pltpu.sync_copy(shb.at[sid], x0)
        xbuf = x0
        nodes = []
        for g in range(GROUPS):
            xv = plsc.load_gather(xbuf, [samps[g], f0])
            nodes.append(jnp.where(xv <= t0, 1, 2))
        for _ in range(1, MAX_DEPTH):
            fs = [plsc.load_gather(feat_v, [nodes[g]]) for g in range(GROUPS)]
            ts = [plsc.load_gather(thr_v, [nodes[g]]) for g in range(GROUPS)]
            xvs = [plsc.load_gather(xbuf, [samps[g], fs[g]])
                   for g in range(GROUPS)]
            for g in range(GROUPS):
                nodes[g] = nodes[g] * 2 + jnp.where(xvs[g] <= ts[g], 1, 2)
        for g in range(GROUPS):
            base = nodes[g] * N_CLASSES - N_INTERNAL * N_CLASSES
            dst = pos10 + 16 * N_CLASSES * g
            for cc in range(N_CLASSES):
                v = plsc.load_gather(val_v, [base + cc])
                plsc.store_scatter(obuf, [dst + cc], v)
        pltpu.sync_copy(obuf, out_hbm.at[pl.ds(c * CHUNK * N_CLASSES,
                                               CHUNK * N_CLASSES)])

    def fire(c, shb, sem):
        pltpu.async_copy(x_hbm.at[pl.ds(c * CHUNK, CHUNK)], shb.at[sid], sem)

    def wait(c, shb, sem):
        pltpu.make_async_copy(x_hbm.at[pl.ds(c * CHUNK, CHUNK)],
                              shb.at[sid], sem).wait()

    fire(wid, sh0, ld0)

    @pl.loop(0, npairs)
    def _(i):
        j = 2 * i
        c_a = wid + NW * j
        c_b = c_a + NW
        c_n = c_b + NW

        @pl.when(c_b < nch)
        def _():
            fire(c_b, sh1, ld1)

        wait(c_a, sh0, ld0)
        process(c_a, sh0, ob0)

        @pl.when(c_n < nch)
        def _():
            fire(c_n, sh0, ld0)

        @pl.when(c_b < nch)
        def _():
            wait(c_b, sh1, ld1)
            process(c_b, sh1, ob1)


def kernel(X, tree_threshold, tree_value, tree_feature, tree_left, tree_right, tree_is_leaf):
    n = X.shape[0]
    del tree_left, tree_right, tree_is_leaf
    val = tree_value[N_INTERNAL:].reshape(N_LEAVES * N_CLASSES)
    f0 = jnp.full((16,), tree_feature[0], jnp.int32)
    t0 = jnp.full((16,), tree_threshold[0], jnp.float32)
    mesh = plsc.VectorSubcoreMesh(
        core_axis_name="c", subcore_axis_name="s", num_cores=NC, num_subcores=NS)
    out = pl.kernel(
        _body,
        out_type=jax.ShapeDtypeStruct((n * N_CLASSES,), jnp.float32),
        mesh=mesh,
        compiler_params=pltpu.CompilerParams(
            needs_layout_passes=False, use_tc_tiling_on_sc=False),
        scratch_types=[
            pltpu.VMEM((N_LEAVES,), jnp.int32),
            pltpu.VMEM((N_LEAVES,), jnp.float32),
            pltpu.VMEM((N_LEAVES * N_CLASSES,), jnp.float32),
            pltpu.VMEM((16,), jnp.int32),
            pltpu.VMEM((16,), jnp.float32),
            pltpu.VMEM_SHARED((NS, CHUNK, NF), jnp.float32),
            pltpu.VMEM_SHARED((NS, CHUNK, NF), jnp.float32),
            pltpu.VMEM((CHUNK, NF), jnp.float32),
            pltpu.VMEM((CHUNK * N_CLASSES,), jnp.float32),
            pltpu.VMEM((CHUNK * N_CLASSES,), jnp.float32),
            pltpu.SemaphoreType.DMA,
            pltpu.SemaphoreType.DMA,
        ],
    )(X, tree_threshold, tree_feature, val, f0, t0)
    return out.reshape(n, N_CLASSES)


# speedup vs baseline: 1.2777x; 1.2777x over previous
import functools

import jax
import jax.numpy as jnp
from jax import lax
from jax.experimental import pallas as pl
from jax.experimental.pallas import tpu as pltpu
from jax.experimental.pallas import tpu_sc as plsc

MAX_DEPTH = 12
N_CLASSES = 10
NF = 128
N_INTERNAL = 2 ** MAX_DEPTH - 1
N_LEAVES = 2 ** MAX_DEPTH
VP = 16
CHUNK = 160
GROUPS = CHUNK // 16
NC, NS = 2, 16
NW = NC * NS


def _body(x_hbm, thr_hbm, feat_hbm, val_hbm, f0_hbm, t0_hbm, out_hbm,
          feat_v, thr_v, f0_v, t0_v, x0, x1, x2, x3, idx0, ob0,
          ld0, ld1, ld2, ld3, gsem):
    n = x_hbm.shape[0]
    nch = n // CHUNK
    njobs = (nch + NW - 1) // NW
    nquads = (njobs + 3) // 4
    wid = lax.axis_index("s") * NC + lax.axis_index("c")
    xbufs = [x0, x1, x2, x3]
    sems = [ld0, ld1, ld2, ld3]

    pltpu.sync_copy(feat_hbm.at[pl.ds(0, N_LEAVES)], feat_v)
    pltpu.sync_copy(thr_hbm.at[pl.ds(0, N_LEAVES)], thr_v)
    pltpu.sync_copy(f0_hbm, f0_v)
    pltpu.sync_copy(t0_hbm, t0_v)
    f0 = f0_v[...]
    t0 = t0_v[...]

    lane = lax.iota(jnp.int32, 16)
    samps = [lane + 16 * g for g in range(GROUPS)]

    def process(c, xbuf, idxbuf, obuf):
        nodes = []
        for g in range(GROUPS):
            xv = plsc.load_gather(xbuf, [samps[g], f0])
            nodes.append(jnp.where(xv <= t0, 1, 2))
        for _ in range(1, MAX_DEPTH):
            fs = [plsc.load_gather(feat_v, [nodes[g]]) for g in range(GROUPS)]
            ts = [plsc.load_gather(thr_v, [nodes[g]]) for g in range(GROUPS)]
            xvs = [plsc.load_gather(xbuf, [samps[g], fs[g]])
                   for g in range(GROUPS)]
            for g in range(GROUPS):
                nodes[g] = nodes[g] * 2 + jnp.where(xvs[g] <= ts[g], 1, 2)
        for g in range(GROUPS):
            idxbuf[pl.ds(16 * g, 16)] = nodes[g] - N_INTERNAL
        pltpu.async_copy(val_hbm.at[idxbuf], obuf, gsem).wait()
        pltpu.sync_copy(obuf, out_hbm.at[pl.ds(c * CHUNK, CHUNK)])

    def fire(c, xbuf, sem):
        pltpu.async_copy(x_hbm.at[pl.ds(c * CHUNK, CHUNK)], xbuf, sem)

    fire(wid, x0, ld0)
    fire(wid + NW, x1, ld1)
    fire(wid + 2 * NW, x2, ld2)

    @pl.loop(0, nquads)
    def _(i):
        for b in range(4):
            c = wid + NW * (4 * i + b)
            c_pre = c + 3 * NW

            @pl.when(c_pre < nch)
            def _():
                fire(c_pre, xbufs[(b + 3) % 4], sems[(b + 3) % 4])

            @pl.when(c < nch)
            def _():
                pltpu.make_async_copy(x_hbm.at[pl.ds(c * CHUNK, CHUNK)],
                                      xbufs[b], sems[b]).wait()
                process(c, xbufs[b], idx0, ob0)


def kernel(X, tree_threshold, tree_value, tree_feature, tree_left, tree_right, tree_is_leaf):
    n = X.shape[0]
    del tree_left, tree_right, tree_is_leaf
    val = jnp.pad(tree_value[N_INTERNAL:], ((0, 0), (0, VP - N_CLASSES)))
    f0 = jnp.full((16,), tree_feature[0], jnp.int32)
    t0 = jnp.full((16,), tree_threshold[0], jnp.float32)
    mesh = plsc.VectorSubcoreMesh(
        core_axis_name="c", subcore_axis_name="s", num_cores=NC, num_subcores=NS)
    out = pl.kernel(
        _body,
        out_type=jax.ShapeDtypeStruct((n, VP), jnp.float32),
        mesh=mesh,
        compiler_params=pltpu.CompilerParams(
            needs_layout_passes=False, use_tc_tiling_on_sc=False),
        scratch_types=[
            pltpu.VMEM((N_LEAVES,), jnp.int32),
            pltpu.VMEM((N_LEAVES,), jnp.float32),
            pltpu.VMEM((16,), jnp.int32),
            pltpu.VMEM((16,), jnp.float32),
            pltpu.VMEM((CHUNK, NF), jnp.float32),
            pltpu.VMEM((CHUNK, NF), jnp.float32),
            pltpu.VMEM((CHUNK, NF), jnp.float32),
            pltpu.VMEM((CHUNK, NF), jnp.float32),
            pltpu.VMEM((CHUNK,), jnp.int32),
            pltpu.VMEM((CHUNK, VP), jnp.float32),
            pltpu.SemaphoreType.DMA,
            pltpu.SemaphoreType.DMA,
            pltpu.SemaphoreType.DMA,
            pltpu.SemaphoreType.DMA,
            pltpu.SemaphoreType.DMA,
        ],
    )(X, tree_threshold, tree_feature, val, f0, t0)
    return out[:, :N_CLASSES]


# speedup vs baseline: 1.2803x; 1.0021x over previous
import functools

import jax
import jax.numpy as jnp
from jax import lax
from jax.experimental import pallas as pl
from jax.experimental.pallas import tpu as pltpu
from jax.experimental.pallas import tpu_sc as plsc

MAX_DEPTH = 12
N_CLASSES = 10
NF = 128
N_INTERNAL = 2 ** MAX_DEPTH - 1
N_LEAVES = 2 ** MAX_DEPTH
VP = 16
CHUNK = 400
GROUPS = CHUNK // 16
NC, NS = 2, 16
NW = NC * NS


def _body(x_hbm, thr_hbm, feat_hbm, val_hbm, f0_hbm, t0_hbm, out_hbm,
          feat_v, thr_v, f0_v, t0_v, x0, x1, idx0, idx1, ob0, ob1,
          ld0, ld1, gsem):
    n = x_hbm.shape[0]
    nch = n // CHUNK
    njobs = (nch + NW - 1) // NW
    npairs = (njobs + 1) // 2
    wid = lax.axis_index("s") * NC + lax.axis_index("c")

    pltpu.sync_copy(feat_hbm.at[pl.ds(0, N_LEAVES)], feat_v)
    pltpu.sync_copy(thr_hbm.at[pl.ds(0, N_LEAVES)], thr_v)
    pltpu.sync_copy(f0_hbm, f0_v)
    pltpu.sync_copy(t0_hbm, t0_v)
    f0 = f0_v[...]
    t0 = t0_v[...]

    lane = lax.iota(jnp.int32, 16)
    samps = [lane + 16 * g for g in range(GROUPS)]

    def process(c, xbuf, idxbuf, obuf):
        nodes = []
        for g in range(GROUPS):
            xv = plsc.load_gather(xbuf, [samps[g], f0])
            nodes.append(jnp.where(xv <= t0, 1, 2))
        for _ in range(1, MAX_DEPTH):
            fs = [plsc.load_gather(feat_v, [nodes[g]]) for g in range(GROUPS)]
            ts = [plsc.load_gather(thr_v, [nodes[g]]) for g in range(GROUPS)]
            xvs = [plsc.load_gather(xbuf, [samps[g], fs[g]])
                   for g in range(GROUPS)]
            for g in range(GROUPS):
                nodes[g] = nodes[g] * 2 + jnp.where(xvs[g] <= ts[g], 1, 2)
        for g in range(GROUPS):
            idxbuf[pl.ds(16 * g, 16)] = nodes[g] - N_INTERNAL
        pltpu.async_copy(val_hbm.at[idxbuf], obuf, gsem).wait()
        pltpu.sync_copy(obuf, out_hbm.at[pl.ds(c * CHUNK, CHUNK)])

    def fire(c, xbuf, sem):
        pltpu.async_copy(x_hbm.at[pl.ds(c * CHUNK, CHUNK)], xbuf, sem)

    fire(wid, x0, ld0)

    @pl.loop(0, npairs)
    def _(i):
        j = 2 * i
        c_a = wid + NW * j
        c_b = c_a + NW
        c_n = c_b + NW

        @pl.when(c_b < nch)
        def _():
            fire(c_b, x1, ld1)

        pltpu.make_async_copy(x_hbm.at[pl.ds(c_a * CHUNK, CHUNK)], x0, ld0).wait()
        process(c_a, x0, idx0, ob0)

        @pl.when(c_n < nch)
        def _():
            fire(c_n, x0, ld0)

        @pl.when(c_b < nch)
        def _():
            pltpu.make_async_copy(x_hbm.at[pl.ds(c_b * CHUNK, CHUNK)], x1, ld1).wait()
            process(c_b, x1, idx1, ob1)


def kernel(X, tree_threshold, tree_value, tree_feature, tree_left, tree_right, tree_is_leaf):
    n = X.shape[0]
    del tree_left, tree_right, tree_is_leaf
    val = jnp.pad(tree_value[N_INTERNAL:], ((0, 0), (0, VP - N_CLASSES)))
    f0 = jnp.full((16,), tree_feature[0], jnp.int32)
    t0 = jnp.full((16,), tree_threshold[0], jnp.float32)
    mesh = plsc.VectorSubcoreMesh(
        core_axis_name="c", subcore_axis_name="s", num_cores=NC, num_subcores=NS)
    out = pl.kernel(
        _body,
        out_type=jax.ShapeDtypeStruct((n, VP), jnp.float32),
        mesh=mesh,
        compiler_params=pltpu.CompilerParams(
            needs_layout_passes=False, use_tc_tiling_on_sc=False),
        scratch_types=[
            pltpu.VMEM((N_LEAVES,), jnp.int32),
            pltpu.VMEM((N_LEAVES,), jnp.float32),
            pltpu.VMEM((16,), jnp.int32),
            pltpu.VMEM((16,), jnp.float32),
            pltpu.VMEM((CHUNK, NF), jnp.float32),
            pltpu.VMEM((CHUNK, NF), jnp.float32),
            pltpu.VMEM((CHUNK,), jnp.int32),
            pltpu.VMEM((CHUNK,), jnp.int32),
            pltpu.VMEM((CHUNK, VP), jnp.float32),
            pltpu.VMEM((CHUNK, VP), jnp.float32),
            pltpu.SemaphoreType.DMA,
            pltpu.SemaphoreType.DMA,
            pltpu.SemaphoreType.DMA,
        ],
    )(X, tree_threshold, tree_feature, val, f0, t0)
    return out[:, :N_CLASSES]
